# baseline (speedup 1.0000x reference)
import functools

import jax
import jax.numpy as jnp
from jax import lax
from jax.experimental import pallas as pl
from jax.experimental.pallas import tpu as pltpu
from jax.experimental.pallas import tpu_sc as plsc

_B = 16384
_NNEG = 20
_NPAD = 24
_D = 128
_C2 = 32
_NE = 8
_NBUF = 8
_NBUF3 = 4


def _run_segment(nchunks, nbuf, start, wait_gather, start_stores,
                 wait_stores):
  look = nbuf - 1
  nblocks = nchunks // nbuf

  for g in range(min(look, nchunks)):
    start(g, g % nbuf)

  def position_full(j, b):
    bg = (b + look) % nbuf
    wait_stores(j + look - nbuf, bg)
    start(j + look, bg)
    wait_gather(j, b)
    start_stores(j, b)

  for b in range(min(nbuf, nchunks)):
    j = b
    g = j + look
    if g < nchunks:
      bg = g % nbuf
      if g >= nbuf:
        wait_stores(g - nbuf, bg)
      start(g, bg)
    wait_gather(j, b)
    start_stores(j, b)

  if nblocks >= 3:
    def body(i, carry):
      j0 = i * nbuf
      for b in range(nbuf):
        position_full(j0 + b, b)
      return carry
    lax.fori_loop(1, nblocks - 1, body, 0)

  if nblocks >= 2:
    j0 = (nblocks - 1) * nbuf
    for b in range(nbuf):
      j = j0 + b
      g = j + look
      if g < nchunks:
        bg = g % nbuf
        wait_stores(g - nbuf, bg)
        start(g, bg)
      wait_gather(j, b)
      start_stores(j, b)

  for j in range(max(0, nchunks - nbuf), nchunks):
    wait_stores(j, j % nbuf)


def _seg2d(table, idx_v, out, out_base, nrows, bufs, gsems, ssems):
  nchunks = nrows // _C2

  def start(j, b):
    pltpu.async_copy(table.at[idx_v.at[pl.ds(j * _C2, _C2)]],
                     bufs[b].at[pl.ds(0, _C2)], gsems[b])

  def wait_gather(j, b):
    pltpu.make_async_copy(table.at[idx_v.at[pl.ds(j * _C2, _C2)]],
                          bufs[b].at[pl.ds(0, _C2)], gsems[b]).wait()

  def start_stores(j, b):
    pltpu.async_copy(bufs[b].at[pl.ds(0, _C2)],
                     out.at[pl.ds(out_base + j * _C2, _C2)], ssems[b])

  def wait_stores(j, b):
    pltpu.make_async_copy(bufs[b].at[pl.ds(0, _C2)],
                          out.at[pl.ds(out_base + j * _C2, _C2)],
                          ssems[b]).wait()

  _run_segment(nchunks, len(bufs), start, wait_gather, start_stores,
               wait_stores)


def _seg3d(table, idx_v, out3, ent_base, nent, bufs3, gsems, ssems):
  nchunks = nent // _NE

  def start(j, b):
    for e in range(_NE):
      pltpu.async_copy(
          table.at[idx_v.at[pl.ds((j * _NE + e) * _NPAD, _NNEG)]],
          bufs3[b].at[e], gsems[b])

  def wait_gather(j, b):
    for e in range(_NE):
      pltpu.make_async_copy(
          table.at[idx_v.at[pl.ds((j * _NE + e) * _NPAD, _NNEG)]],
          bufs3[b].at[e], gsems[b]).wait()

  def start_stores(j, b):
    pltpu.async_copy(bufs3[b], out3.at[pl.ds(ent_base + j * _NE, _NE)],
                     ssems[b])

  def wait_stores(j, b):
    pltpu.make_async_copy(bufs3[b], out3.at[pl.ds(ent_base + j * _NE, _NE)],
                          ssems[b]).wait()

  _run_segment(nchunks, len(bufs3), start, wait_gather, start_stores,
               wait_stores)


def _make_sc_gather():
  info = plsc.get_sparse_core_info()
  nc, ns = info.num_cores, info.num_subcores
  nw = nc * ns
  bt = _B // nw
  be = _B // nw
  bn = be * _NNEG
  mesh = plsc.VectorSubcoreMesh(core_axis_name="c", subcore_axis_name="s")

  @functools.partial(
      pl.kernel,
      mesh=mesh,
      out_type=(
          jax.ShapeDtypeStruct((_B, _D), jnp.float32),
          jax.ShapeDtypeStruct((_B, _D), jnp.float32),
          jax.ShapeDtypeStruct((_B, _NNEG, _D), jnp.float32),
      ),
      scratch_types=[
          pltpu.VMEM((bt,), jnp.int32),
          pltpu.VMEM((bt,), jnp.int32),
          pltpu.VMEM((be * _NPAD,), jnp.int32),
      ] + [pltpu.VMEM((_C2, _D), jnp.float32) for _ in range(4)]
        + [pltpu.VMEM((_NE, _NNEG, _D), jnp.float32) for _ in range(_NBUF3)]
        + [pltpu.SemaphoreType.DMA for _ in range(2 * _NBUF)],
      compiler_params=pltpu.CompilerParams(use_tc_tiling_on_sc=True),
  )
  def sc_gather(tgt_hbm, ctx_hbm, noise_hbm, in_emb, out_emb,
                out_t, out_c, out_n,
                idx_t, idx_c, idx_n,
                c0, c1, c2, c3,
                b0, b1, b2, b3,
                g0, g1, g2, g3, g4, g5, g6, g7,
                s0, s1, s2, s3, s4, s5, s6, s7):
    wid = lax.axis_index("s") * nc + lax.axis_index("c")
    pltpu.sync_copy(tgt_hbm.at[pl.ds(wid * bt, bt)], idx_t)
    pltpu.sync_copy(ctx_hbm.at[pl.ds(wid * bt, bt)], idx_c)
    pltpu.sync_copy(noise_hbm.at[pl.ds(wid * be * _NPAD, be * _NPAD)], idx_n)
    bufs2 = (c0, c1, c2, c3)
    bufs3 = (b0, b1, b2, b3)
    gsems = (g0, g1, g2, g3, g4, g5, g6, g7)
    ssems = (s0, s1, s2, s3, s4, s5, s6, s7)
    _seg2d(in_emb, idx_t, out_t, wid * bt, bt, bufs2, gsems[:4], ssems[:4])
    _seg2d(out_emb, idx_c, out_c, wid * bt, bt, bufs2, gsems[:4], ssems[:4])
    _seg3d(out_emb, idx_n, out_n, wid * be, be, bufs3, gsems[:4], ssems[:4])

  return sc_gather


_sc_gather = _make_sc_gather()


def kernel(target, context, noise, input_embedding, output_embedding):
  noise_pad = jnp.pad(noise.astype(jnp.int32), ((0, 0), (0, _NPAD - _NNEG)))
  out_t, out_c, out_n = _sc_gather(
      target.astype(jnp.int32),
      context.astype(jnp.int32),
      noise_pad.reshape(-1),
      input_embedding,
      output_embedding,
  )
  return out_t, out_c, out_n


# speedup vs baseline: 1.8914x; 1.8914x over previous
import functools

import jax
import jax.numpy as jnp
from jax import lax
from jax.experimental import pallas as pl
from jax.experimental.pallas import tpu as pltpu
from jax.experimental.pallas import tpu_sc as plsc

_B = 16384
_NNEG = 20
_D = 128
_C2 = 64
_CN = 128
_NBUF = 4


def _run_segment(nchunks, nbuf, start, wait_gather, start_stores,
                 wait_stores):
  look = nbuf - 1
  nblocks = nchunks // nbuf

  for g in range(min(look, nchunks)):
    start(g, g % nbuf)

  def position_full(j, b):
    bg = (b + look) % nbuf
    wait_stores(j + look - nbuf, bg)
    start(j + look, bg)
    wait_gather(j, b)
    start_stores(j, b)

  for b in range(min(nbuf, nchunks)):
    j = b
    g = j + look
    if g < nchunks:
      bg = g % nbuf
      if g >= nbuf:
        wait_stores(g - nbuf, bg)
      start(g, bg)
    wait_gather(j, b)
    start_stores(j, b)

  if nblocks >= 3:
    def body(i, carry):
      j0 = i * nbuf
      for b in range(nbuf):
        position_full(j0 + b, b)
      return carry
    lax.fori_loop(1, nblocks - 1, body, 0)

  if nblocks >= 2:
    j0 = (nblocks - 1) * nbuf
    for b in range(nbuf):
      j = j0 + b
      g = j + look
      if g < nchunks:
        bg = g % nbuf
        wait_stores(g - nbuf, bg)
        start(g, bg)
      wait_gather(j, b)
      start_stores(j, b)

  for j in range(max(0, nchunks - nbuf), nchunks):
    wait_stores(j, j % nbuf)


def _seg(table, idx_v, out, out_base, nrows, chunk, bufs, gsems, ssems):
  nchunks = nrows // chunk

  def start(j, b):
    pltpu.async_copy(table.at[idx_v.at[pl.ds(j * chunk, chunk)]],
                     bufs[b].at[pl.ds(0, chunk)], gsems[b])

  def wait_gather(j, b):
    pltpu.make_async_copy(table.at[idx_v.at[pl.ds(j * chunk, chunk)]],
                          bufs[b].at[pl.ds(0, chunk)], gsems[b]).wait()

  def start_stores(j, b):
    pltpu.async_copy(bufs[b].at[pl.ds(0, chunk)],
                     out.at[pl.ds(out_base + j * chunk, chunk)], ssems[b])

  def wait_stores(j, b):
    pltpu.make_async_copy(bufs[b].at[pl.ds(0, chunk)],
                          out.at[pl.ds(out_base + j * chunk, chunk)],
                          ssems[b]).wait()

  _run_segment(nchunks, len(bufs), start, wait_gather, start_stores,
               wait_stores)


def _make_sc_gather():
  info = plsc.get_sparse_core_info()
  nc, ns = info.num_cores, info.num_subcores
  nw = nc * ns
  bt = _B // nw
  bn = (_B * _NNEG) // nw
  mesh = plsc.VectorSubcoreMesh(core_axis_name="c", subcore_axis_name="s")

  @functools.partial(
      pl.kernel,
      mesh=mesh,
      out_type=(
          jax.ShapeDtypeStruct((_B, _D), jnp.float32),
          jax.ShapeDtypeStruct((_B, _D), jnp.float32),
          jax.ShapeDtypeStruct((_B * _NNEG, _D), jnp.float32),
      ),
      scratch_types=[
          pltpu.VMEM((bt,), jnp.int32),
          pltpu.VMEM((bt,), jnp.int32),
          pltpu.VMEM((bn,), jnp.int32),
      ] + [pltpu.VMEM((_CN, _D), jnp.float32) for _ in range(_NBUF)]
        + [pltpu.SemaphoreType.DMA for _ in range(2 * _NBUF)],
      compiler_params=pltpu.CompilerParams(use_tc_tiling_on_sc=True),
  )
  def sc_gather(tgt_hbm, ctx_hbm, noise_hbm, in_emb, out_emb,
                out_t, out_c, out_n,
                idx_t, idx_c, idx_n,
                b0, b1, b2, b3,
                g0, g1, g2, g3, s0, s1, s2, s3):
    wid = lax.axis_index("s") * nc + lax.axis_index("c")
    pltpu.sync_copy(tgt_hbm.at[pl.ds(wid * bt, bt)], idx_t)
    pltpu.sync_copy(ctx_hbm.at[pl.ds(wid * bt, bt)], idx_c)
    pltpu.sync_copy(noise_hbm.at[pl.ds(wid * bn, bn)], idx_n)
    bufs = (b0, b1, b2, b3)
    gsems = (g0, g1, g2, g3)
    ssems = (s0, s1, s2, s3)
    _seg(in_emb, idx_t, out_t, wid * bt, bt, _C2, bufs, gsems, ssems)
    _seg(out_emb, idx_c, out_c, wid * bt, bt, _C2, bufs, gsems, ssems)
    _seg(out_emb, idx_n, out_n, wid * bn, bn, _CN, bufs, gsems, ssems)

  return sc_gather


_sc_gather = _make_sc_gather()


def kernel(target, context, noise, input_embedding, output_embedding):
  noise_t = jnp.transpose(noise.astype(jnp.int32)).reshape(-1)
  out_t, out_c, out_n = _sc_gather(
      target.astype(jnp.int32),
      context.astype(jnp.int32),
      noise_t,
      input_embedding,
      output_embedding,
  )
  return (out_t, out_c,
          jnp.transpose(out_n.reshape(_NNEG, _B, _D), (1, 0, 2)))


# speedup vs baseline: 1.9021x; 1.0057x over previous
import functools

import jax
import jax.numpy as jnp
from jax import lax
from jax.experimental import pallas as pl
from jax.experimental.pallas import tpu as pltpu
from jax.experimental.pallas import tpu_sc as plsc

_B = 16384
_NNEG = 20
_D = 128
_C2 = 128
_CN = 128
_NBUF = 5


def _run_segment(nchunks, nbuf, start, wait_gather, start_stores,
                 wait_stores):
  look = nbuf - 1
  nblocks = nchunks // nbuf

  for g in range(min(look, nchunks)):
    start(g, g % nbuf)

  def position_full(j, b):
    bg = (b + look) % nbuf
    wait_stores(j + look - nbuf, bg)
    start(j + look, bg)
    wait_gather(j, b)
    start_stores(j, b)

  for b in range(min(nbuf, nchunks)):
    j = b
    g = j + look
    if g < nchunks:
      bg = g % nbuf
      if g >= nbuf:
        wait_stores(g - nbuf, bg)
      start(g, bg)
    wait_gather(j, b)
    start_stores(j, b)

  if nblocks >= 3:
    def body(i, carry):
      j0 = i * nbuf
      for b in range(nbuf):
        position_full(j0 + b, b)
      return carry
    lax.fori_loop(1, nblocks - 1, body, 0)

  if nblocks >= 2:
    j0 = (nblocks - 1) * nbuf
    for b in range(nbuf):
      j = j0 + b
      g = j + look
      if g < nchunks:
        bg = g % nbuf
        wait_stores(g - nbuf, bg)
        start(g, bg)
      wait_gather(j, b)
      start_stores(j, b)

  for j in range(max(0, nchunks - nbuf), nchunks):
    wait_stores(j, j % nbuf)


def _seg(table, idx_v, out, out_base, nrows, chunk, bufs, gsems, ssems):
  nchunks = nrows // chunk

  def start(j, b):
    pltpu.async_copy(table.at[idx_v.at[pl.ds(j * chunk, chunk)]],
                     bufs[b].at[pl.ds(0, chunk)], gsems[b])

  def wait_gather(j, b):
    pltpu.make_async_copy(table.at[idx_v.at[pl.ds(j * chunk, chunk)]],
                          bufs[b].at[pl.ds(0, chunk)], gsems[b]).wait()

  def start_stores(j, b):
    pltpu.async_copy(bufs[b].at[pl.ds(0, chunk)],
                     out.at[pl.ds(out_base + j * chunk, chunk)], ssems[b])

  def wait_stores(j, b):
    pltpu.make_async_copy(bufs[b].at[pl.ds(0, chunk)],
                          out.at[pl.ds(out_base + j * chunk, chunk)],
                          ssems[b]).wait()

  _run_segment(nchunks, len(bufs), start, wait_gather, start_stores,
               wait_stores)


def _make_sc_gather():
  info = plsc.get_sparse_core_info()
  nc, ns = info.num_cores, info.num_subcores
  nw = nc * ns
  bt = _B // nw
  bn = (_B * _NNEG) // nw
  mesh = plsc.VectorSubcoreMesh(core_axis_name="c", subcore_axis_name="s")

  @functools.partial(
      pl.kernel,
      mesh=mesh,
      out_type=(
          jax.ShapeDtypeStruct((_B, _D), jnp.float32),
          jax.ShapeDtypeStruct((_B, _D), jnp.float32),
          jax.ShapeDtypeStruct((_B * _NNEG, _D), jnp.float32),
      ),
      scratch_types=[
          pltpu.VMEM((bt,), jnp.int32),
          pltpu.VMEM((bt,), jnp.int32),
          pltpu.VMEM((bn,), jnp.int32),
      ] + [pltpu.VMEM((_CN, _D), jnp.float32) for _ in range(_NBUF)]
        + [pltpu.SemaphoreType.DMA for _ in range(2 * _NBUF)],
      compiler_params=pltpu.CompilerParams(use_tc_tiling_on_sc=True),
  )
  def sc_gather(tgt_hbm, ctx_hbm, noise_hbm, in_emb, out_emb,
                out_t, out_c, out_n,
                idx_t, idx_c, idx_n,
                b0, b1, b2, b3, b4,
                g0, g1, g2, g3, g4, s0, s1, s2, s3, s4):
    wid = lax.axis_index("s") * nc + lax.axis_index("c")
    pltpu.sync_copy(tgt_hbm.at[pl.ds(wid * bt, bt)], idx_t)
    pltpu.sync_copy(ctx_hbm.at[pl.ds(wid * bt, bt)], idx_c)
    pltpu.sync_copy(noise_hbm.at[pl.ds(wid * bn, bn)], idx_n)
    bufs = (b0, b1, b2, b3, b4)
    gsems = (g0, g1, g2, g3, g4)
    ssems = (s0, s1, s2, s3, s4)
    _seg(in_emb, idx_t, out_t, wid * bt, bt, _C2, bufs[:4], gsems[:4],
         ssems[:4])
    _seg(out_emb, idx_c, out_c, wid * bt, bt, _C2, bufs[:4], gsems[:4],
         ssems[:4])
    _seg(out_emb, idx_n, out_n, wid * bn, bn, _CN, bufs, gsems, ssems)

  return sc_gather


_sc_gather = _make_sc_gather()


def kernel(target, context, noise, input_embedding, output_embedding):
  noise_t = jnp.transpose(noise.astype(jnp.int32)).reshape(-1)
  out_t, out_c, out_n = _sc_gather(
      target.astype(jnp.int32),
      context.astype(jnp.int32),
      noise_t,
      input_embedding,
      output_embedding,
  )
  return (out_t, out_c,
          jnp.transpose(out_n.reshape(_NNEG, _B, _D), (1, 0, 2)))
